# baseline (speedup 1.0000x reference)
import functools

import jax
import jax.numpy as jnp
from jax import lax
from jax.experimental import pallas as pl
from jax.experimental.pallas import tpu as pltpu
from jax.experimental.pallas import tpu_sc as plsc

VOCAB = 100000
PAD_ID = VOCAB - 2
INTEREST_ID = VOCAB - 1

B, S, H = 1024, 200, 64
LANES = 16
S_PAD = 208
N_CHUNKS = S_PAD // LANES


def _tc_weights_kernel(h_ref, ids_ref, w_ref, b_ref, out_ref):
    h = h_ref[...]
    k = h[:, 0:1, :]
    f = jnp.tanh(h + k)
    w = w_ref[...]
    scores = jnp.sum(f * w[None, None, :], axis=-1) + b_ref[0]
    ids = ids_ref[...]
    mask = (ids != PAD_ID) & (ids != INTEREST_ID)
    scores = jnp.where(mask, scores, -jnp.inf)
    out_ref[...] = jax.nn.softmax(scores, axis=-1)


def _tc_weights(hidden_states, input_ids, w_attn, b_attn):
    R = 128
    grid = (B // R,)
    return pl.pallas_call(
        _tc_weights_kernel,
        grid=grid,
        in_specs=[
            pl.BlockSpec((R, S, H), lambda i: (i, 0, 0)),
            pl.BlockSpec((R, S), lambda i: (i, 0)),
            pl.BlockSpec((H,), lambda i: (0,)),
            pl.BlockSpec((1,), lambda i: (0,)),
        ],
        out_specs=pl.BlockSpec((R, S), lambda i: (i, 0)),
        out_shape=jax.ShapeDtypeStruct((B, S), jnp.float32),
    )(hidden_states, input_ids, w_attn, b_attn)


def _sc_scatter(input_ids, weights):
    info = plsc.get_sparse_core_info()
    nc, ns = info.num_cores, info.num_subcores
    nw = nc * ns
    rows_per_w = B // nw

    mesh = plsc.VectorSubcoreMesh(core_axis_name="c", subcore_axis_name="s")

    @functools.partial(
        pl.kernel,
        mesh=mesh,
        out_type=jax.ShapeDtypeStruct((B * VOCAB,), jnp.float32),
        scratch_types=[
            pltpu.VMEM((VOCAB,), jnp.float32),
            pltpu.VMEM((S_PAD,), jnp.int32),
            pltpu.VMEM((S_PAD,), jnp.float32),
        ],
        compiler_params=pltpu.CompilerParams(needs_layout_passes=False),
    )
    def sc_kernel(ids_hbm, w_hbm, out_hbm, row_buf, ids_v, w_v):
        wid = lax.axis_index("s") * nc + lax.axis_index("c")
        zero16f = jnp.zeros((LANES,), jnp.float32)

        def zero_body(i, _):
            row_buf[pl.ds(i * LANES, LANES)] = zero16f
            return _

        lax.fori_loop(0, VOCAB // LANES, zero_body, None)

        ids_v[pl.ds(S_PAD - LANES, LANES)] = jnp.zeros((LANES,), jnp.int32)
        w_v[pl.ds(S_PAD - LANES, LANES)] = zero16f

        def row_body(r, _):
            b = wid * rows_per_w + r
            pltpu.sync_copy(ids_hbm.at[pl.ds(b * S, S)], ids_v.at[pl.ds(0, S)])
            pltpu.sync_copy(w_hbm.at[pl.ds(b * S, S)], w_v.at[pl.ds(0, S)])
            idxs = []
            for j in range(N_CHUNKS):
                idx = ids_v[pl.ds(j * LANES, LANES)]
                w = w_v[pl.ds(j * LANES, LANES)]
                plsc.addupdate_scatter(row_buf, [idx], w)
                idxs.append(idx)
            pltpu.sync_copy(row_buf, out_hbm.at[pl.ds(b * VOCAB, VOCAB)])
            for idx in idxs:
                plsc.store_scatter(row_buf, [idx], zero16f)
            return _

        lax.fori_loop(0, rows_per_w, row_body, None)

    flat = sc_kernel(input_ids.reshape(-1), weights.reshape(-1))
    return flat.reshape(B, VOCAB)


def kernel(hidden_states, input_ids, w_attn, b_attn):
    weights = _tc_weights(hidden_states, input_ids, w_attn, b_attn)
    return _sc_scatter(input_ids, weights)


# speedup vs baseline: 1.6548x; 1.6548x over previous
import functools

import jax
import jax.numpy as jnp
from jax import lax
from jax.experimental import pallas as pl
from jax.experimental.pallas import tpu as pltpu
from jax.experimental.pallas import tpu_sc as plsc

VOCAB = 100000
PAD_ID = VOCAB - 2
INTEREST_ID = VOCAB - 1

B, S, H = 1024, 200, 64
LANES = 16
N_CHUNKS = (S + LANES - 1) // LANES

SHARD_SHIFT = 13
SHARD_C = 1 << SHARD_SHIFT
N_FULL_SHARDS = VOCAB // SHARD_C
TAIL_C = VOCAB - N_FULL_SHARDS * SHARD_C
ROWS_PER_GROUP = 8


def _tc_weights_kernel(h_ref, ids_ref, w_ref, b_ref, out_ref):
    h = h_ref[...]
    k = h[:, 0:1, :]
    f = jnp.tanh(h + k)
    w = w_ref[...]
    scores = jnp.sum(f * w[None, None, :], axis=-1) + b_ref[0]
    ids = ids_ref[...]
    mask = (ids != PAD_ID) & (ids != INTEREST_ID)
    scores = jnp.where(mask, scores, -jnp.inf)
    out_ref[...] = jax.nn.softmax(scores, axis=-1)


def _tc_weights(hidden_states, input_ids, w_attn, b_attn):
    R = 128
    grid = (B // R,)
    return pl.pallas_call(
        _tc_weights_kernel,
        grid=grid,
        in_specs=[
            pl.BlockSpec((R, S, H), lambda i: (i, 0, 0)),
            pl.BlockSpec((R, S), lambda i: (i, 0)),
            pl.BlockSpec((H,), lambda i: (0,)),
            pl.BlockSpec((1,), lambda i: (0,)),
        ],
        out_specs=pl.BlockSpec((R, S), lambda i: (i, 0)),
        out_shape=jax.ShapeDtypeStruct((B, S), jnp.float32),
    )(hidden_states, input_ids, w_attn, b_attn)


def _sc_scatter(input_ids, weights):
    info = plsc.get_sparse_core_info()
    nc, ns = info.num_cores, info.num_subcores
    nw = nc * ns
    n_groups = B // ROWS_PER_GROUP
    groups_per_w = n_groups // nw
    g_words = ROWS_PER_GROUP * S
    stage = g_words + LANES

    mesh = plsc.VectorSubcoreMesh(core_axis_name="c", subcore_axis_name="s")

    @functools.partial(
        pl.kernel,
        mesh=mesh,
        out_type=jax.ShapeDtypeStruct((B, VOCAB), jnp.float32),
        scratch_types=[
            pltpu.VMEM((ROWS_PER_GROUP, SHARD_C), jnp.float32),
            pltpu.VMEM((ROWS_PER_GROUP, TAIL_C), jnp.float32),
            pltpu.VMEM((stage,), jnp.int32),
            pltpu.VMEM((stage,), jnp.float32),
        ],
        compiler_params=pltpu.CompilerParams(needs_layout_passes=False),
    )
    def sc_kernel(ids_hbm, w_hbm, out_hbm, buf, buft, ids_v, w_v):
        wid = lax.axis_index("s") * nc + lax.axis_index("c")
        zero16f = jnp.zeros((LANES,), jnp.float32)
        lane8 = lax.iota(jnp.int32, LANES) < (S - (N_CHUNKS - 1) * LANES)

        for r in range(ROWS_PER_GROUP):
            def zb(i, _, r=r):
                buf[r, pl.ds(i * LANES, LANES)] = zero16f
                return _
            lax.fori_loop(0, SHARD_C // LANES, zb, None)
            def zbt(i, _, r=r):
                buft[r, pl.ds(i * LANES, LANES)] = zero16f
                return _
            lax.fori_loop(0, TAIL_C // LANES, zbt, None)

        def shard_pass(s, cols, acc):
            for r in range(ROWS_PER_GROUP):
                r_splat = jnp.full((LANES,), r, jnp.int32)
                for cj in range(N_CHUNKS):
                    c0 = r * S + cj * LANES
                    ids = ids_v[pl.ds(c0, LANES)]
                    m = (ids >> SHARD_SHIFT) == s
                    if cj == N_CHUNKS - 1:
                        m = m & lane8
                    local = ids & (SHARD_C - 1)
                    if acc:
                        w = w_v[pl.ds(c0, LANES)]
                        plsc.addupdate_scatter(cols, [r_splat, local], w, mask=m)
                    else:
                        plsc.store_scatter(cols, [r_splat, local], zero16f, mask=m)

        def group_body(g, _):
            gg = wid * groups_per_w + g
            row0 = gg * ROWS_PER_GROUP
            pltpu.sync_copy(ids_hbm.at[pl.ds(gg * g_words, g_words)],
                            ids_v.at[pl.ds(0, g_words)])
            pltpu.sync_copy(w_hbm.at[pl.ds(gg * g_words, g_words)],
                            w_v.at[pl.ds(0, g_words)])

            def shard_body(s, _):
                shard_pass(s, buf, True)
                pltpu.sync_copy(
                    buf, out_hbm.at[pl.ds(row0, ROWS_PER_GROUP),
                                    pl.ds(s * SHARD_C, SHARD_C)])
                shard_pass(s, buf, False)
                return _

            lax.fori_loop(0, N_FULL_SHARDS, shard_body, None)

            shard_pass(N_FULL_SHARDS, buft, True)
            pltpu.sync_copy(
                buft, out_hbm.at[pl.ds(row0, ROWS_PER_GROUP),
                                 pl.ds(N_FULL_SHARDS * SHARD_C, TAIL_C)])
            shard_pass(N_FULL_SHARDS, buft, False)
            return _

        lax.fori_loop(0, groups_per_w, group_body, None)

    return sc_kernel(input_ids.reshape(-1), weights.reshape(-1))


def kernel(hidden_states, input_ids, w_attn, b_attn):
    weights = _tc_weights(hidden_states, input_ids, w_attn, b_attn)
    return _sc_scatter(input_ids, weights)


# speedup vs baseline: 1.8713x; 1.1308x over previous
import functools

import jax
import jax.numpy as jnp
from jax import lax
from jax.experimental import pallas as pl
from jax.experimental.pallas import tpu as pltpu
from jax.experimental.pallas import tpu_sc as plsc

VOCAB = 100000
PAD_ID = VOCAB - 2
INTEREST_ID = VOCAB - 1

B, S, H = 1024, 200, 64
LANES = 16
N_CHUNKS = (S + LANES - 1) // LANES

SHARD_SHIFT = 13
SHARD_C = 1 << SHARD_SHIFT
N_FULL_SHARDS = VOCAB // SHARD_C
TAIL_C = VOCAB - N_FULL_SHARDS * SHARD_C
ROWS_PER_GROUP = 8


def _tc_weights_kernel(h_ref, ids_ref, w_ref, b_ref, out_ref):
    h = h_ref[...]
    k = h[0:1]
    f = jnp.tanh(h + k)
    w = w_ref[...]
    scores = jnp.sum(f * w[None, :, None], axis=1) + b_ref[0]
    ids = ids_ref[...]
    mask = (ids != PAD_ID) & (ids != INTEREST_ID)
    scores = jnp.where(mask, scores, -jnp.inf)
    out_ref[...] = jax.nn.softmax(scores, axis=0).T


def _tc_weights(hidden_states, input_ids, w_attn, b_attn):
    h_t = jnp.transpose(hidden_states, (1, 2, 0))
    ids_t = jnp.transpose(input_ids, (1, 0))
    R = 128
    grid = (B // R,)
    return pl.pallas_call(
        _tc_weights_kernel,
        grid=grid,
        in_specs=[
            pl.BlockSpec((S, H, R), lambda i: (0, 0, i)),
            pl.BlockSpec((S, R), lambda i: (0, i)),
            pl.BlockSpec((H,), lambda i: (0,)),
            pl.BlockSpec((1,), lambda i: (0,)),
        ],
        out_specs=pl.BlockSpec((R, S), lambda i: (i, 0)),
        out_shape=jax.ShapeDtypeStruct((B, S), jnp.float32),
    )(h_t, ids_t, w_attn, b_attn)


def _sc_scatter(input_ids, weights):
    info = plsc.get_sparse_core_info()
    nc, ns = info.num_cores, info.num_subcores
    nw = nc * ns
    n_groups = B // ROWS_PER_GROUP
    groups_per_w = n_groups // nw
    g_words = ROWS_PER_GROUP * S
    stage = g_words + LANES

    mesh = plsc.VectorSubcoreMesh(core_axis_name="c", subcore_axis_name="s")

    @functools.partial(
        pl.kernel,
        mesh=mesh,
        out_type=jax.ShapeDtypeStruct((B, VOCAB), jnp.float32),
        scratch_types=[
            pltpu.VMEM((ROWS_PER_GROUP, SHARD_C), jnp.float32),
            pltpu.VMEM((ROWS_PER_GROUP, TAIL_C), jnp.float32),
            pltpu.VMEM((stage,), jnp.int32),
            pltpu.VMEM((stage,), jnp.float32),
        ],
        compiler_params=pltpu.CompilerParams(needs_layout_passes=False),
    )
    def sc_kernel(ids_hbm, w_hbm, out_hbm, buf, buft, ids_v, w_v):
        wid = lax.axis_index("s") * nc + lax.axis_index("c")
        zero16f = jnp.zeros((LANES,), jnp.float32)
        lane8 = lax.iota(jnp.int32, LANES) < (S - (N_CHUNKS - 1) * LANES)

        for r in range(ROWS_PER_GROUP):
            def zb(i, _, r=r):
                buf[r, pl.ds(i * LANES, LANES)] = zero16f
                return _
            lax.fori_loop(0, SHARD_C // LANES, zb, None)
            def zbt(i, _, r=r):
                buft[r, pl.ds(i * LANES, LANES)] = zero16f
                return _
            lax.fori_loop(0, TAIL_C // LANES, zbt, None)

        def shard_pass(s, cols, acc):
            for r in range(ROWS_PER_GROUP):
                r_splat = jnp.full((LANES,), r, jnp.int32)
                for cj in range(N_CHUNKS):
                    c0 = r * S + cj * LANES
                    ids = ids_v[pl.ds(c0, LANES)]
                    m = (ids >> SHARD_SHIFT) == s
                    if cj == N_CHUNKS - 1:
                        m = m & lane8
                    local = ids & (SHARD_C - 1)
                    if acc:
                        w = w_v[pl.ds(c0, LANES)]
                        plsc.addupdate_scatter(cols, [r_splat, local], w, mask=m)
                    else:
                        plsc.store_scatter(cols, [r_splat, local], zero16f, mask=m)

        def group_body(g, _):
            gg = wid * groups_per_w + g
            row0 = gg * ROWS_PER_GROUP
            pltpu.sync_copy(ids_hbm.at[pl.ds(gg * g_words, g_words)],
                            ids_v.at[pl.ds(0, g_words)])
            pltpu.sync_copy(w_hbm.at[pl.ds(gg * g_words, g_words)],
                            w_v.at[pl.ds(0, g_words)])

            def shard_body(s, _):
                shard_pass(s, buf, True)
                pltpu.sync_copy(
                    buf, out_hbm.at[pl.ds(row0, ROWS_PER_GROUP),
                                    pl.ds(s * SHARD_C, SHARD_C)])
                shard_pass(s, buf, False)
                return _

            lax.fori_loop(0, N_FULL_SHARDS, shard_body, None)

            shard_pass(N_FULL_SHARDS, buft, True)
            pltpu.sync_copy(
                buft, out_hbm.at[pl.ds(row0, ROWS_PER_GROUP),
                                 pl.ds(N_FULL_SHARDS * SHARD_C, TAIL_C)])
            shard_pass(N_FULL_SHARDS, buft, False)
            return _

        lax.fori_loop(0, groups_per_w, group_body, None)

    return sc_kernel(input_ids.reshape(-1), weights.reshape(-1))


def kernel(hidden_states, input_ids, w_attn, b_attn):
    weights = _tc_weights(hidden_states, input_ids, w_attn, b_attn)
    return _sc_scatter(input_ids, weights)
